# baseline (speedup 1.0000x reference)
import functools

import jax
import jax.numpy as jnp
from jax import lax
from jax.experimental import pallas as pl
from jax.experimental.pallas import tpu as pltpu
from jax.experimental.pallas import tpu_sc as plsc

N = 10000
E = 320000
NP = 10240
NTILES = 32
NCHUNK = 80
EPT = NCHUNK * 128
ROWS_PT = NP // 16
DUMMY = N
PHASE = 40
K_SPLIT = (160, 0)

_mesh = plsc.VectorSubcoreMesh(core_axis_name="c", subcore_axis_name="s")


def _deg_parts(dst2):
  @functools.partial(
      pl.kernel,
      out_type=jax.ShapeDtypeStruct((NTILES, NP), jnp.float32),
      mesh=_mesh,
      compiler_params=pltpu.CompilerParams(needs_layout_passes=False),
      scratch_types=[
          pltpu.VMEM((EPT,), jnp.int32),
          pltpu.VMEM((NP,), jnp.float32),
      ],
  )
  def k(dst_hbm, out_hbm, dstv, hist):
    c = lax.axis_index("c")
    s = lax.axis_index("s")
    wid = c * 16 + s
    pltpu.sync_copy(dst_hbm.at[wid], dstv)
    zeros = jnp.zeros((16,), jnp.float32)
    ones = jnp.ones((16,), jnp.float32)

    def zbody(i, carry):
      hist[pl.ds(i * 16, 16)] = zeros
      return carry

    lax.fori_loop(0, NP // 16, zbody, 0)

    def body(i, carry):
      idx = dstv[pl.ds(i * 16, 16)]
      plsc.addupdate_scatter(hist, [idx], ones)
      return carry

    lax.fori_loop(0, EPT // 16, body, 0)
    pltpu.sync_copy(hist, out_hbm.at[wid])

  return k(dst2)


def _edge_scatter(g, src2, dst2, d, k_by_core):
  k0, k1 = k_by_core
  assert k0 % PHASE == 0 and k1 % PHASE == 0 and k0 + k1 == 2 * NCHUNK
  max_phases = max(k0, k1) // PHASE

  @functools.partial(
      pl.kernel,
      out_type=jax.ShapeDtypeStruct((2, NP, d), jnp.float32),
      mesh=_mesh,
      scratch_types=[
          pltpu.VMEM((PHASE, 128), jnp.int32),
          pltpu.VMEM((PHASE, 128), jnp.int32),
          pltpu.VMEM((128, d), jnp.float32),
          pltpu.VMEM((128, d), jnp.float32),
          pltpu.VMEM_SHARED((NP, d), jnp.float32),
          pltpu.SemaphoreType.DMA,
          pltpu.SemaphoreType.DMA,
      ],
  )
  def k(g_hbm, src_hbm, dst_hbm, zz_hbm, out_hbm,
        srcv, dstv, bufA, bufB, acc, semA, semB):
    c = lax.axis_index("c")
    s = lax.axis_index("s")
    r0 = s * ROWS_PT
    pltpu.sync_copy(zz_hbm, acc.at[pl.ds(r0, ROWS_PT)])
    plsc.subcore_barrier()

    my_k = jnp.where(c == 0, k0, k1)
    base = jnp.where(c == 0, s * k0, 16 * k0 + s * k1)
    n_phases = my_k // PHASE

    for h in range(max_phases):

      @pl.when(h < n_phases)
      def _phase():
        c0 = base + h * PHASE
        pltpu.sync_copy(src_hbm.at[pl.ds(c0, PHASE)], srcv)
        pltpu.sync_copy(dst_hbm.at[pl.ds(c0, PHASE)], dstv)

        pltpu.async_copy(g_hbm.at[srcv.at[0]], bufA, semA)
        pltpu.async_copy(g_hbm.at[srcv.at[1]], bufB, semB)

        def body(i, carry):
          jA = 2 * i
          jB = 2 * i + 1
          pltpu.make_async_copy(g_hbm.at[srcv.at[0]], bufA, semA).wait()
          pltpu.sync_copy(bufA, acc.at[dstv.at[jA]], add=True)
          nA = jnp.minimum(jA + 2, PHASE - 2)
          pltpu.async_copy(g_hbm.at[srcv.at[nA]], bufA, semA)
          pltpu.make_async_copy(g_hbm.at[srcv.at[1]], bufB, semB).wait()
          pltpu.sync_copy(bufB, acc.at[dstv.at[jB]], add=True)
          nB = jnp.minimum(jB + 2, PHASE - 1)
          pltpu.async_copy(g_hbm.at[srcv.at[nB]], bufB, semB)
          return carry

        lax.fori_loop(0, PHASE // 2, body, 0)
        pltpu.make_async_copy(g_hbm.at[srcv.at[0]], bufA, semA).wait()
        pltpu.make_async_copy(g_hbm.at[srcv.at[1]], bufB, semB).wait()

    plsc.subcore_barrier()
    pltpu.sync_copy(acc.at[pl.ds(r0, ROWS_PT)],
                    out_hbm.at[c, pl.ds(r0, ROWS_PT)])

  return k(g, src2, dst2, jnp.zeros((ROWS_PT, d), jnp.float32))


def _dinv(deg_parts):
  def body(dp_ref, o_ref):
    deg = jnp.sum(dp_ref[...], axis=0) + 1.0
    o_ref[...] = lax.rsqrt(deg)[:, None]

  return pl.pallas_call(
      body,
      out_shape=jax.ShapeDtypeStruct((NP, 1), jnp.float32),
  )(deg_parts)


def _tc_scale_matmul(x, w, dinv):
  din, dout = w.shape

  def body(x_ref, w_ref, dv_ref, o_ref):
    h = jnp.dot(x_ref[...], w_ref[...], preferred_element_type=jnp.float32)
    o_ref[...] = h * dv_ref[...]

  return pl.pallas_call(
      body,
      grid=(10,),
      in_specs=[
          pl.BlockSpec((1000, din), lambda i: (i, 0)),
          pl.BlockSpec((din, dout), lambda i: (0, 0)),
          pl.BlockSpec((1000, 1), lambda i: (i, 0)),
      ],
      out_specs=pl.BlockSpec((1000, dout), lambda i: (i, 0)),
      out_shape=jax.ShapeDtypeStruct((N, dout), jnp.float32),
  )(x, w, dinv)


def _tc_combine_relu_matmul(sp, g, dinv, b, w):
  din, dout = w.shape

  def body(sp_ref, g_ref, dv_ref, b_ref, w_ref, o_ref):
    ssum = sp_ref[0] + sp_ref[1] + g_ref[...]
    a = ssum * dv_ref[...] + b_ref[...]
    r = jnp.maximum(a, 0.0)
    h = jnp.dot(r, w_ref[...], preferred_element_type=jnp.float32)
    o_ref[...] = h * dv_ref[...]

  return pl.pallas_call(
      body,
      grid=(10,),
      in_specs=[
          pl.BlockSpec((2, 1000, din), lambda i: (0, i, 0)),
          pl.BlockSpec((1000, din), lambda i: (i, 0)),
          pl.BlockSpec((1000, 1), lambda i: (i, 0)),
          pl.BlockSpec((1, din), lambda i: (0, 0)),
          pl.BlockSpec((din, dout), lambda i: (0, 0)),
      ],
      out_specs=pl.BlockSpec((1000, dout), lambda i: (i, 0)),
      out_shape=jax.ShapeDtypeStruct((N, dout), jnp.float32),
  )(sp, g, dinv, b, w)


def _tc_combine_logsoftmax(sp, g, dinv, b, dout):
  dpad = g.shape[1]

  def body(sp_ref, g_ref, dv_ref, b_ref, o_ref):
    full = (sp_ref[0] + sp_ref[1] + g_ref[...]) * dv_ref[...]
    o = full[:, :dout] + b_ref[...]
    m = jnp.max(o, axis=1, keepdims=True)
    e = jnp.exp(o - m)
    lse = jnp.log(jnp.sum(e, axis=1, keepdims=True))
    o_ref[...] = o - m - lse

  return pl.pallas_call(
      body,
      grid=(10,),
      in_specs=[
          pl.BlockSpec((2, 1000, dpad), lambda i: (0, i, 0)),
          pl.BlockSpec((1000, dpad), lambda i: (i, 0)),
          pl.BlockSpec((1000, 1), lambda i: (i, 0)),
          pl.BlockSpec((1, dout), lambda i: (0, 0)),
      ],
      out_specs=pl.BlockSpec((1000, dout), lambda i: (i, 0)),
      out_shape=jax.ShapeDtypeStruct((N, dout), jnp.float32),
  )(sp, g, dinv, b)


def kernel(x, edge_index, W1, b1, W2, b2):
  src = edge_index[0].astype(jnp.int32)
  dst = edge_index[1].astype(jnp.int32)
  pad = NTILES * EPT - E
  src_p = jnp.concatenate([src, jnp.zeros((pad,), jnp.int32)])
  dst_p = jnp.concatenate([dst, jnp.full((pad,), DUMMY, jnp.int32)])
  src2 = src_p.reshape(NTILES * NCHUNK, 128)
  dst2 = dst_p.reshape(NTILES * NCHUNK, 128)
  dsth = dst_p.reshape(NTILES, EPT)

  dparts = _deg_parts(dsth)
  dinv = _dinv(dparts)

  g1 = _tc_scale_matmul(x, W1, dinv)
  s1 = _edge_scatter(g1, src2, dst2, W1.shape[1], K_SPLIT)
  W2p = jnp.pad(W2, ((0, 0), (0, 128 - W2.shape[1])))
  g2 = _tc_combine_relu_matmul(s1, g1, dinv, b1.reshape(1, -1), W2p)
  s2 = _edge_scatter(g2, src2, dst2, 128, K_SPLIT)
  return _tc_combine_logsoftmax(s2, g2, dinv, b2.reshape(1, -1), W2.shape[1])


# speedup vs baseline: 1.0450x; 1.0450x over previous
import functools

import jax
import jax.numpy as jnp
from jax import lax
from jax.experimental import pallas as pl
from jax.experimental.pallas import tpu as pltpu
from jax.experimental.pallas import tpu_sc as plsc

N = 10000
E = 320000
NP = 10240
NTILES = 32
EP = 327680
PB = 5120
NPH = EP // PB
DUMMY = N

_mesh = plsc.VectorSubcoreMesh(core_axis_name="c", subcore_axis_name="s")
_params = pltpu.CompilerParams(needs_layout_passes=False)


def _deg_parts(dst2):
  ept = EP // NTILES

  @functools.partial(
      pl.kernel,
      out_type=jax.ShapeDtypeStruct((NTILES, NP), jnp.float32),
      mesh=_mesh,
      compiler_params=_params,
      scratch_types=[
          pltpu.VMEM((ept,), jnp.int32),
          pltpu.VMEM((NP,), jnp.float32),
      ],
  )
  def k(dst_hbm, out_hbm, dstv, hist):
    c = lax.axis_index("c")
    s = lax.axis_index("s")
    wid = c * 16 + s
    pltpu.sync_copy(dst_hbm.at[wid], dstv)
    zeros = jnp.zeros((16,), jnp.float32)
    ones = jnp.ones((16,), jnp.float32)

    def zbody(i, carry):
      hist[pl.ds(i * 16, 16)] = zeros
      return carry

    lax.fori_loop(0, NP // 16, zbody, 0)

    def body(i, carry):
      idx = dstv[pl.ds(i * 16, 16)]
      plsc.addupdate_scatter(hist, [idx], ones)
      return carry

    lax.fori_loop(0, ept // 16, body, 0)
    pltpu.sync_copy(hist, out_hbm.at[wid])

  return k(dst2)


def _edge_scatter_t(gt, src1, dst1):
  d = gt.shape[0]
  cpt = d // NTILES

  @functools.partial(
      pl.kernel,
      out_type=jax.ShapeDtypeStruct((d, NP), jnp.float32),
      mesh=_mesh,
      compiler_params=_params,
      scratch_types=[
          pltpu.VMEM((PB,), jnp.int32),
          pltpu.VMEM((PB,), jnp.int32),
          pltpu.VMEM((PB,), jnp.int32),
          pltpu.VMEM((PB,), jnp.int32),
          pltpu.VMEM((cpt, NP), jnp.float32),
          pltpu.VMEM((cpt, NP), jnp.float32),
          pltpu.SemaphoreType.DMA,
          pltpu.SemaphoreType.DMA,
      ],
  )
  def k(gt_hbm, src_hbm, dst_hbm, out_hbm,
        srcA, dstA, srcB, dstB, slab, acc, semA, semB):
    c = lax.axis_index("c")
    s = lax.axis_index("s")
    r0 = (c * 16 + s) * cpt
    pltpu.sync_copy(gt_hbm.at[pl.ds(r0, cpt)], slab)

    zeros = jnp.zeros((16,), jnp.float32)

    def zbody(i, carry):
      for cc in range(cpt):
        acc[cc, pl.ds(i * 16, 16)] = zeros
      return carry

    lax.fori_loop(0, NP // 16, zbody, 0)

    def start(ph, sv, dv, sem):
      e0 = ph * PB
      pltpu.async_copy(src_hbm.at[pl.ds(e0, PB)], sv, sem)
      pltpu.async_copy(dst_hbm.at[pl.ds(e0, PB)], dv, sem)

    def wait(sv, dv, sem):
      pltpu.make_async_copy(src_hbm.at[pl.ds(0, PB)], sv, sem).wait()
      pltpu.make_async_copy(dst_hbm.at[pl.ds(0, PB)], dv, sem).wait()

    def process(sv, dv):
      cvs = [jnp.full((16,), cc, jnp.int32) for cc in range(cpt)]

      def ibody(i, carry):
        s16 = sv[pl.ds(i * 16, 16)]
        d16 = dv[pl.ds(i * 16, 16)]
        for cc in range(cpt):
          val = plsc.load_gather(slab, [cvs[cc], s16])
          plsc.addupdate_scatter(acc, [cvs[cc], d16], val)
        return carry

      lax.fori_loop(0, PB // 16, ibody, 0)

    start(0, srcA, dstA, semA)
    start(1, srcB, dstB, semB)

    def phases(i, carry):
      phA = 2 * i
      phB = 2 * i + 1
      wait(srcA, dstA, semA)
      process(srcA, dstA)
      start(jnp.minimum(phA + 2, NPH - 2), srcA, dstA, semA)
      wait(srcB, dstB, semB)
      process(srcB, dstB)
      start(jnp.minimum(phB + 2, NPH - 1), srcB, dstB, semB)
      return carry

    lax.fori_loop(0, NPH // 2, phases, 0)
    wait(srcA, dstA, semA)
    wait(srcB, dstB, semB)

    pltpu.sync_copy(acc, out_hbm.at[pl.ds(r0, cpt)])

  return k(gt, src1, dst1)


def _dinv2(deg_parts):
  def body(dp_ref, oc_ref, or_ref):
    deg = jnp.sum(dp_ref[...], axis=0) + 1.0
    dv = lax.rsqrt(deg)
    oc_ref[...] = dv[:, None]
    or_ref[...] = dv[None, :]

  return pl.pallas_call(
      body,
      out_shape=[
          jax.ShapeDtypeStruct((NP, 1), jnp.float32),
          jax.ShapeDtypeStruct((1, NP), jnp.float32),
      ],
  )(deg_parts)


def _tc_scale_matmul_t(xp, w, dinv):
  din, dout = w.shape

  def body(x_ref, w_ref, dv_ref, o_ref):
    h = jnp.dot(x_ref[...], w_ref[...], preferred_element_type=jnp.float32)
    o_ref[...] = jnp.transpose(h * dv_ref[...])

  return pl.pallas_call(
      body,
      grid=(16,),
      in_specs=[
          pl.BlockSpec((640, din), lambda i: (i, 0)),
          pl.BlockSpec((din, dout), lambda i: (0, 0)),
          pl.BlockSpec((640, 1), lambda i: (i, 0)),
      ],
      out_specs=pl.BlockSpec((dout, 640), lambda i: (0, i)),
      out_shape=jax.ShapeDtypeStruct((dout, NP), jnp.float32),
  )(xp, w, dinv)


def _tc_combine_relu_matmul_t(st, gt, dinvr, bc, wt):
  dout, din = wt.shape

  def body(st_ref, gt_ref, dv_ref, b_ref, w_ref, o_ref):
    a = (st_ref[...] + gt_ref[...]) * dv_ref[...] + b_ref[...]
    r = jnp.maximum(a, 0.0)
    h = jnp.dot(w_ref[...], r, preferred_element_type=jnp.float32)
    o_ref[...] = h * dv_ref[...]

  return pl.pallas_call(
      body,
      grid=(16,),
      in_specs=[
          pl.BlockSpec((din, 640), lambda i: (0, i)),
          pl.BlockSpec((din, 640), lambda i: (0, i)),
          pl.BlockSpec((1, 640), lambda i: (0, i)),
          pl.BlockSpec((din, 1), lambda i: (0, 0)),
          pl.BlockSpec((dout, din), lambda i: (0, 0)),
      ],
      out_specs=pl.BlockSpec((dout, 640), lambda i: (0, i)),
      out_shape=jax.ShapeDtypeStruct((dout, NP), jnp.float32),
  )(st, gt, dinvr, bc, wt)


def _tc_combine_logsoftmax_t(st, gt, dinvr, bc):
  d = gt.shape[0]

  def body(st_ref, gt_ref, dv_ref, b_ref, o_ref):
    o = (st_ref[...] + gt_ref[...]) * dv_ref[...] + b_ref[...]
    m = jnp.max(o, axis=0, keepdims=True)
    e = jnp.exp(o - m)
    lse = jnp.log(jnp.sum(e, axis=0, keepdims=True))
    o_ref[...] = jnp.transpose(o - m - lse)

  return pl.pallas_call(
      body,
      grid=(16,),
      in_specs=[
          pl.BlockSpec((d, 640), lambda i: (0, i)),
          pl.BlockSpec((d, 640), lambda i: (0, i)),
          pl.BlockSpec((1, 640), lambda i: (0, i)),
          pl.BlockSpec((d, 1), lambda i: (0, 0)),
      ],
      out_specs=pl.BlockSpec((640, d), lambda i: (i, 0)),
      out_shape=jax.ShapeDtypeStruct((NP, d), jnp.float32),
  )(st, gt, dinvr, bc)


def kernel(x, edge_index, W1, b1, W2, b2):
  src = edge_index[0].astype(jnp.int32)
  dst = edge_index[1].astype(jnp.int32)
  pad = EP - E
  src1 = jnp.concatenate([src, jnp.zeros((pad,), jnp.int32)])
  dst1 = jnp.concatenate([dst, jnp.full((pad,), DUMMY, jnp.int32)])
  dsth = dst1.reshape(NTILES, EP // NTILES)

  dparts = _deg_parts(dsth)
  dinvc, dinvr = _dinv2(dparts)

  xp = jnp.pad(x, ((0, NP - N), (0, 0)))
  g1t = _tc_scale_matmul_t(xp, W1, dinvc)
  s1t = _edge_scatter_t(g1t, src1, dst1)
  g2t = _tc_combine_relu_matmul_t(s1t, g1t, dinvr, b1.reshape(-1, 1), W2.T)
  s2t = _edge_scatter_t(g2t, src1, dst1)
  out = _tc_combine_logsoftmax_t(s2t, g2t, dinvr, b2.reshape(-1, 1))
  return out[:N]


# speedup vs baseline: 2.4082x; 2.3045x over previous
import functools

import jax
import jax.numpy as jnp
from jax import lax
from jax.experimental import pallas as pl
from jax.experimental.pallas import tpu as pltpu
from jax.experimental.pallas import tpu_sc as plsc

N = 10000
E = 320000
NP = 10240
NTILES = 32
EP = 327680
PB = 5120
NPH = EP // PB
DUMMY = N

_mesh = plsc.VectorSubcoreMesh(core_axis_name="c", subcore_axis_name="s")
_params = pltpu.CompilerParams(needs_layout_passes=False)


def _deg_parts(dst2):
  ept = EP // NTILES

  @functools.partial(
      pl.kernel,
      out_type=jax.ShapeDtypeStruct((NTILES, NP), jnp.float32),
      mesh=_mesh,
      compiler_params=_params,
      scratch_types=[
          pltpu.VMEM((ept,), jnp.int32),
          pltpu.VMEM((NP,), jnp.float32),
      ],
  )
  def k(dst_hbm, out_hbm, dstv, hist):
    c = lax.axis_index("c")
    s = lax.axis_index("s")
    wid = c * 16 + s
    pltpu.sync_copy(dst_hbm.at[wid], dstv)
    zeros = jnp.zeros((16,), jnp.float32)
    ones = jnp.ones((16,), jnp.float32)

    def zbody(i, carry):
      hist[pl.ds(i * 16, 16)] = zeros
      return carry

    lax.fori_loop(0, NP // 16, zbody, 0)

    def body(i, carry):
      idx = dstv[pl.ds(i * 16, 16)]
      plsc.addupdate_scatter(hist, [idx], ones)
      return carry

    lax.fori_loop(0, ept // 16, body, 0)
    pltpu.sync_copy(hist, out_hbm.at[wid])

  return k(dst2)


def _edge_scatter_t(gt, src1, dst1):
  d = gt.shape[0]
  cpt = d // NTILES

  @functools.partial(
      pl.kernel,
      out_type=jax.ShapeDtypeStruct((d, NP), jnp.float32),
      mesh=_mesh,
      compiler_params=_params,
      scratch_types=[
          pltpu.VMEM((PB,), jnp.int32),
          pltpu.VMEM((PB,), jnp.int32),
          pltpu.VMEM((PB,), jnp.int32),
          pltpu.VMEM((PB,), jnp.int32),
          pltpu.VMEM((cpt, NP), jnp.float32),
          pltpu.VMEM((cpt, NP), jnp.float32),
          pltpu.SemaphoreType.DMA,
          pltpu.SemaphoreType.DMA,
      ],
  )
  def k(gt_hbm, src_hbm, dst_hbm, out_hbm,
        srcA, dstA, srcB, dstB, slab, acc, semA, semB):
    c = lax.axis_index("c")
    s = lax.axis_index("s")
    r0 = (c * 16 + s) * cpt
    pltpu.sync_copy(gt_hbm.at[pl.ds(r0, cpt)], slab)

    zeros = jnp.zeros((16,), jnp.float32)

    def zbody(i, carry):
      for cc in range(cpt):
        acc[cc, pl.ds(i * 16, 16)] = zeros
      return carry

    lax.fori_loop(0, NP // 16, zbody, 0)

    def start(ph, sv, dv, sem):
      e0 = ph * PB
      pltpu.async_copy(src_hbm.at[pl.ds(e0, PB)], sv, sem)
      pltpu.async_copy(dst_hbm.at[pl.ds(e0, PB)], dv, sem)

    def wait(sv, dv, sem):
      pltpu.make_async_copy(src_hbm.at[pl.ds(0, PB)], sv, sem).wait()
      pltpu.make_async_copy(dst_hbm.at[pl.ds(0, PB)], dv, sem).wait()

    def process(sv, dv):
      cvs = [jnp.full((16,), cc, jnp.int32) for cc in range(cpt)]

      @plsc.parallel_loop(0, PB // 16, unroll=4)
      def ibody(i):
        s16 = sv[pl.ds(i * 16, 16)]
        d16 = dv[pl.ds(i * 16, 16)]
        for cc in range(cpt):
          val = plsc.load_gather(slab, [cvs[cc], s16])
          plsc.addupdate_scatter(acc, [cvs[cc], d16], val)

    start(0, srcA, dstA, semA)
    start(1, srcB, dstB, semB)

    def phases(i, carry):
      phA = 2 * i
      phB = 2 * i + 1
      wait(srcA, dstA, semA)
      process(srcA, dstA)
      start(jnp.minimum(phA + 2, NPH - 2), srcA, dstA, semA)
      wait(srcB, dstB, semB)
      process(srcB, dstB)
      start(jnp.minimum(phB + 2, NPH - 1), srcB, dstB, semB)
      return carry

    lax.fori_loop(0, NPH // 2, phases, 0)
    wait(srcA, dstA, semA)
    wait(srcB, dstB, semB)

    pltpu.sync_copy(acc, out_hbm.at[pl.ds(r0, cpt)])

  return k(gt, src1, dst1)


def _dinv2(deg_parts):
  def body(dp_ref, oc_ref, or_ref):
    deg = jnp.sum(dp_ref[...], axis=0) + 1.0
    dv = lax.rsqrt(deg)
    oc_ref[...] = dv[:, None]
    or_ref[...] = dv[None, :]

  return pl.pallas_call(
      body,
      out_shape=[
          jax.ShapeDtypeStruct((NP, 1), jnp.float32),
          jax.ShapeDtypeStruct((1, NP), jnp.float32),
      ],
  )(deg_parts)


def _tc_scale_matmul_t(xp, w, dinv):
  din, dout = w.shape

  def body(x_ref, w_ref, dv_ref, o_ref):
    h = jnp.dot(x_ref[...], w_ref[...], preferred_element_type=jnp.float32)
    o_ref[...] = jnp.transpose(h * dv_ref[...])

  return pl.pallas_call(
      body,
      grid=(16,),
      in_specs=[
          pl.BlockSpec((640, din), lambda i: (i, 0)),
          pl.BlockSpec((din, dout), lambda i: (0, 0)),
          pl.BlockSpec((640, 1), lambda i: (i, 0)),
      ],
      out_specs=pl.BlockSpec((dout, 640), lambda i: (0, i)),
      out_shape=jax.ShapeDtypeStruct((dout, NP), jnp.float32),
  )(xp, w, dinv)


def _tc_combine_relu_matmul_t(st, gt, dinvr, bc, wt):
  dout, din = wt.shape

  def body(st_ref, gt_ref, dv_ref, b_ref, w_ref, o_ref):
    a = (st_ref[...] + gt_ref[...]) * dv_ref[...] + b_ref[...]
    r = jnp.maximum(a, 0.0)
    h = jnp.dot(w_ref[...], r, preferred_element_type=jnp.float32)
    o_ref[...] = h * dv_ref[...]

  return pl.pallas_call(
      body,
      grid=(16,),
      in_specs=[
          pl.BlockSpec((din, 640), lambda i: (0, i)),
          pl.BlockSpec((din, 640), lambda i: (0, i)),
          pl.BlockSpec((1, 640), lambda i: (0, i)),
          pl.BlockSpec((din, 1), lambda i: (0, 0)),
          pl.BlockSpec((dout, din), lambda i: (0, 0)),
      ],
      out_specs=pl.BlockSpec((dout, 640), lambda i: (0, i)),
      out_shape=jax.ShapeDtypeStruct((dout, NP), jnp.float32),
  )(st, gt, dinvr, bc, wt)


def _tc_combine_logsoftmax_t(st, gt, dinvr, bc):
  d = gt.shape[0]

  def body(st_ref, gt_ref, dv_ref, b_ref, o_ref):
    o = (st_ref[...] + gt_ref[...]) * dv_ref[...] + b_ref[...]
    m = jnp.max(o, axis=0, keepdims=True)
    e = jnp.exp(o - m)
    lse = jnp.log(jnp.sum(e, axis=0, keepdims=True))
    o_ref[...] = jnp.transpose(o - m - lse)

  return pl.pallas_call(
      body,
      grid=(16,),
      in_specs=[
          pl.BlockSpec((d, 640), lambda i: (0, i)),
          pl.BlockSpec((d, 640), lambda i: (0, i)),
          pl.BlockSpec((1, 640), lambda i: (0, i)),
          pl.BlockSpec((d, 1), lambda i: (0, 0)),
      ],
      out_specs=pl.BlockSpec((640, d), lambda i: (i, 0)),
      out_shape=jax.ShapeDtypeStruct((NP, d), jnp.float32),
  )(st, gt, dinvr, bc)


def kernel(x, edge_index, W1, b1, W2, b2):
  src = edge_index[0].astype(jnp.int32)
  dst = edge_index[1].astype(jnp.int32)
  pad = EP - E
  src1 = jnp.concatenate([src, jnp.zeros((pad,), jnp.int32)])
  dst1 = jnp.concatenate([dst, jnp.full((pad,), DUMMY, jnp.int32)])
  dsth = dst1.reshape(NTILES, EP // NTILES)

  dparts = _deg_parts(dsth)
  dinvc, dinvr = _dinv2(dparts)

  xp = jnp.pad(x, ((0, NP - N), (0, 0)))
  g1t = _tc_scale_matmul_t(xp, W1, dinvc)
  s1t = _edge_scatter_t(g1t, src1, dst1)
  g2t = _tc_combine_relu_matmul_t(s1t, g1t, dinvr, b1.reshape(-1, 1), W2.T)
  s2t = _edge_scatter_t(g2t, src1, dst1)
  out = _tc_combine_logsoftmax_t(s2t, g2t, dinvr, b2.reshape(-1, 1))
  return out[:N]
